# baseline (speedup 1.0000x reference)
import jax
import jax.numpy as jnp
from jax.experimental import pallas as pl
from jax.experimental.pallas import tpu as pltpu

_BINS = 64
_WIN = 16
_BS = 8


def _shift_right(x, d):
    bs = x.shape[0]
    return jnp.concatenate([jnp.zeros((bs, d), x.dtype), x[:, :-d]], axis=1)


def _rhythm_kernel(mel_ref, uniform_ref, target_ref, stats_ref, trace_ref):
    mel = mel_ref[...]
    bs, T, M = mel.shape
    inv_m = 1.0 / M
    inv_t = 1.0 / T

    melT = jnp.transpose(mel, (0, 2, 1))
    prev = jnp.concatenate([melT[:, :, :1], melT[:, :, :-1]], axis=2)
    energy = jnp.sum(melT, axis=1) * inv_m
    flux = jnp.sum(jnp.abs(melT - prev), axis=1) * inv_m

    mean_energy = jnp.sum(energy, axis=1, keepdims=True) * inv_t
    speech = (energy > mean_energy).astype(jnp.float32)
    pause = 1.0 - speech

    tot_s = jnp.sum(speech, axis=1, keepdims=True)
    voiced_ratio = tot_s * inv_t
    tot_p = jnp.sum(pause, axis=1, keepdims=True)
    pause_ratio = tot_p * inv_t

    prev_s = _shift_right(speech, 1)
    n_runs_s = jnp.sum(speech * (1.0 - prev_s), axis=1, keepdims=True)
    mean_speech_frames = jnp.where(
        n_runs_s > 0.0, tot_s / jnp.maximum(n_runs_s, 1.0), 0.0)

    prev_p = _shift_right(pause, 1)
    n_runs_p = jnp.sum(pause * (1.0 - prev_p), axis=1, keepdims=True)
    mean_pause_frames = jnp.where(
        n_runs_p > 0.0, tot_p / jnp.maximum(n_runs_p, 1.0), 0.0)

    csum = speech
    d = 1
    while d < T:
        csum = csum + _shift_right(csum, d)
        d *= 2

    local_rate = (csum - _shift_right(csum, _WIN)) * (1.0 / _WIN)
    half = T // 2
    rate_trend = (jnp.sum(local_rate[:, half:], axis=1, keepdims=True)
                  - jnp.sum(local_rate[:, :half], axis=1, keepdims=True)) * (1.0 / half)

    mean_flux = jnp.sum(flux, axis=1, keepdims=True) * inv_t
    boundary_strength = flux / (mean_flux + 1e-6)
    boundary_ratio = jnp.sum(
        (boundary_strength > 1.5).astype(jnp.float32), axis=1, keepdims=True) * inv_t

    prog_speech = csum / jnp.maximum(tot_s, 1.0)
    uniform = uniform_ref[...]
    progress = jnp.where(tot_s > 0.0, prog_speech, uniform)
    seg_bias = progress - uniform

    stats_ref[:, 0, :] = jnp.concatenate(
        [pause_ratio, mean_pause_frames, mean_speech_frames,
         rate_trend, boundary_ratio, voiced_ratio], axis=1)

    target = target_ref[...]
    prog3 = progress[:, None, :]
    cmp = (prog3 < target).astype(jnp.float32)
    right = jnp.sum(cmp, axis=2, keepdims=True)
    left_i = jnp.clip(right - 1.0, 0.0, T - 1.0).astype(jnp.int32)

    iota_t = jax.lax.broadcasted_iota(jnp.int32, (bs, T, _BINS), 1)
    left_row = jnp.transpose(left_i, (0, 2, 1))
    onehot_lt = (iota_t == left_row).astype(jnp.float32)

    chans = jnp.stack(
        [pause, local_rate, boundary_strength, seg_bias, speech],
        axis=1)
    chans_next = jnp.concatenate(
        [chans[:, :, 1:], chans[:, :, -1:]], axis=2)
    both = jnp.concatenate([chans, chans_next], axis=1)
    gathered = jax.lax.dot_general(
        both, onehot_lt, (((2,), (1,)), ((0,), (0,))),
        preferred_element_type=jnp.float32)
    fl = jnp.transpose(gathered[:, :5, :], (0, 2, 1))
    fr = jnp.transpose(gathered[:, 5:, :], (0, 2, 1))

    lp = jnp.max(cmp * prog3, axis=2, keepdims=True)
    rp = jnp.min(prog3 + cmp * 2.0, axis=2, keepdims=True)
    denom = jnp.maximum(jnp.abs(rp - lp), 1e-6)
    alpha = jnp.clip((target - lp) / denom, 0.0, 1.0)
    interp = fl[:, :, :5] * (1.0 - alpha) + fr[:, :, :5] * alpha

    f_first = jnp.concatenate(
        [pause[:, :1], local_rate[:, :1], boundary_strength[:, :1],
         seg_bias[:, :1], speech[:, :1]], axis=1)[:, None, :]
    f_last = jnp.concatenate(
        [pause[:, -1:], local_rate[:, -1:], boundary_strength[:, -1:],
         seg_bias[:, -1:], speech[:, -1:]], axis=1)[:, None, :]
    out = jnp.where(right <= 0.0, f_first, interp)
    out = jnp.where(right >= float(T), f_last, out)
    trace_ref[...] = out


def kernel(ref_mel):
    B, T, M = ref_mel.shape
    uniform = jnp.linspace(0.0, 1.0, T, dtype=jnp.float32).reshape(1, T)
    target = jnp.linspace(0.0, 1.0, _BINS, dtype=jnp.float32).reshape(1, _BINS, 1)
    stats, trace = pl.pallas_call(
        _rhythm_kernel,
        grid=(B // _BS,),
        in_specs=[
            pl.BlockSpec((_BS, T, M), lambda g: (g, 0, 0)),
            pl.BlockSpec((1, T), lambda g: (0, 0)),
            pl.BlockSpec((1, _BINS, 1), lambda g: (0, 0, 0)),
        ],
        out_specs=[
            pl.BlockSpec((_BS, 1, 6), lambda g: (g, 0, 0)),
            pl.BlockSpec((_BS, _BINS, 5), lambda g: (g, 0, 0)),
        ],
        out_shape=[
            jax.ShapeDtypeStruct((B, 1, 6), jnp.float32),
            jax.ShapeDtypeStruct((B, _BINS, 5), jnp.float32),
        ],
        compiler_params=pltpu.CompilerParams(
            dimension_semantics=("parallel",)),
    )(ref_mel, uniform, target)
    return stats.reshape(B, 6), trace


# speedup vs baseline: 1.0605x; 1.0605x over previous
import jax
import jax.numpy as jnp
from jax.experimental import pallas as pl
from jax.experimental.pallas import tpu as pltpu

_BINS = 64
_WIN = 16
_BS = 4


def _shift_right(x, d):
    bs = x.shape[0]
    return jnp.concatenate([jnp.zeros((bs, d), x.dtype), x[:, :-d]], axis=1)


def _rhythm_kernel(mel_ref, uniform_ref, target_ref, stats_ref, trace_ref):
    mel = mel_ref[...]
    bs, T, M = mel.shape
    inv_m = 1.0 / M
    inv_t = 1.0 / T

    melT = jnp.transpose(mel, (0, 2, 1))
    prev = jnp.concatenate([melT[:, :, :1], melT[:, :, :-1]], axis=2)
    energy = jnp.sum(melT, axis=1) * inv_m
    flux = jnp.sum(jnp.abs(melT - prev), axis=1) * inv_m

    mean_energy = jnp.sum(energy, axis=1, keepdims=True) * inv_t
    speech = (energy > mean_energy).astype(jnp.float32)
    pause = 1.0 - speech

    tot_s = jnp.sum(speech, axis=1, keepdims=True)
    voiced_ratio = tot_s * inv_t
    tot_p = jnp.sum(pause, axis=1, keepdims=True)
    pause_ratio = tot_p * inv_t

    prev_s = _shift_right(speech, 1)
    n_runs_s = jnp.sum(speech * (1.0 - prev_s), axis=1, keepdims=True)
    mean_speech_frames = jnp.where(
        n_runs_s > 0.0, tot_s / jnp.maximum(n_runs_s, 1.0), 0.0)

    prev_p = _shift_right(pause, 1)
    n_runs_p = jnp.sum(pause * (1.0 - prev_p), axis=1, keepdims=True)
    mean_pause_frames = jnp.where(
        n_runs_p > 0.0, tot_p / jnp.maximum(n_runs_p, 1.0), 0.0)

    csum = speech
    d = 1
    while d < T:
        csum = csum + _shift_right(csum, d)
        d *= 2

    local_rate = (csum - _shift_right(csum, _WIN)) * (1.0 / _WIN)
    half = T // 2
    rate_trend = (jnp.sum(local_rate[:, half:], axis=1, keepdims=True)
                  - jnp.sum(local_rate[:, :half], axis=1, keepdims=True)) * (1.0 / half)

    mean_flux = jnp.sum(flux, axis=1, keepdims=True) * inv_t
    boundary_strength = flux / (mean_flux + 1e-6)
    boundary_ratio = jnp.sum(
        (boundary_strength > 1.5).astype(jnp.float32), axis=1, keepdims=True) * inv_t

    prog_speech = csum / jnp.maximum(tot_s, 1.0)
    uniform = uniform_ref[...]
    progress = jnp.where(tot_s > 0.0, prog_speech, uniform)
    seg_bias = progress - uniform

    stats_ref[:, 0, :] = jnp.concatenate(
        [pause_ratio, mean_pause_frames, mean_speech_frames,
         rate_trend, boundary_ratio, voiced_ratio], axis=1)

    target = target_ref[...]
    prog3 = progress[:, None, :]
    cmp = (prog3 < target).astype(jnp.float32)
    right = jnp.sum(cmp, axis=2, keepdims=True)
    left_i = jnp.clip(right - 1.0, 0.0, T - 1.0).astype(jnp.int32)

    iota_t = jax.lax.broadcasted_iota(jnp.int32, (bs, T, _BINS), 1)
    left_row = jnp.transpose(left_i, (0, 2, 1))
    onehot_lt = (iota_t == left_row).astype(jnp.float32)

    p_hi = progress.astype(jnp.bfloat16).astype(jnp.float32)
    p_rem = progress - p_hi
    p_mid = p_rem.astype(jnp.bfloat16).astype(jnp.float32)
    p_lo = p_rem - p_mid
    chans = jnp.stack(
        [pause, local_rate, boundary_strength, seg_bias, speech,
         p_hi, p_mid, p_lo],
        axis=1)
    chans_next = jnp.concatenate(
        [chans[:, :, 1:], chans[:, :, -1:]], axis=2)
    both = jnp.concatenate([chans, chans_next], axis=1)
    gathered = jax.lax.dot_general(
        both, onehot_lt, (((2,), (1,)), ((0,), (0,))),
        preferred_element_type=jnp.float32)
    fl = jnp.transpose(gathered[:, :5, :], (0, 2, 1))
    fr = jnp.transpose(gathered[:, 8:13, :], (0, 2, 1))

    lp = (gathered[:, 5, :] + gathered[:, 6, :]
          + gathered[:, 7, :])[:, :, None]
    rp = (gathered[:, 13, :] + gathered[:, 14, :]
          + gathered[:, 15, :])[:, :, None]
    denom = jnp.maximum(jnp.abs(rp - lp), 1e-6)
    alpha = jnp.clip((target - lp) / denom, 0.0, 1.0)
    interp = fl[:, :, :5] * (1.0 - alpha) + fr[:, :, :5] * alpha

    f_first = jnp.concatenate(
        [pause[:, :1], local_rate[:, :1], boundary_strength[:, :1],
         seg_bias[:, :1], speech[:, :1]], axis=1)[:, None, :]
    f_last = jnp.concatenate(
        [pause[:, -1:], local_rate[:, -1:], boundary_strength[:, -1:],
         seg_bias[:, -1:], speech[:, -1:]], axis=1)[:, None, :]
    out = jnp.where(right <= 0.0, f_first, interp)
    out = jnp.where(right >= float(T), f_last, out)
    trace_ref[...] = out


def kernel(ref_mel):
    B, T, M = ref_mel.shape
    uniform = jnp.linspace(0.0, 1.0, T, dtype=jnp.float32).reshape(1, T)
    target = jnp.linspace(0.0, 1.0, _BINS, dtype=jnp.float32).reshape(1, _BINS, 1)
    stats, trace = pl.pallas_call(
        _rhythm_kernel,
        grid=(B // _BS,),
        in_specs=[
            pl.BlockSpec((_BS, T, M), lambda g: (g, 0, 0)),
            pl.BlockSpec((1, T), lambda g: (0, 0)),
            pl.BlockSpec((1, _BINS, 1), lambda g: (0, 0, 0)),
        ],
        out_specs=[
            pl.BlockSpec((_BS, 1, 6), lambda g: (g, 0, 0)),
            pl.BlockSpec((_BS, _BINS, 5), lambda g: (g, 0, 0)),
        ],
        out_shape=[
            jax.ShapeDtypeStruct((B, 1, 6), jnp.float32),
            jax.ShapeDtypeStruct((B, _BINS, 5), jnp.float32),
        ],
        compiler_params=pltpu.CompilerParams(
            dimension_semantics=("parallel",)),
    )(ref_mel, uniform, target)
    return stats.reshape(B, 6), trace
